# baseline (speedup 1.0000x reference)
import functools

import jax
import jax.numpy as jnp
from jax import lax
from jax.experimental import pallas as pl
from jax.experimental.pallas import tpu as pltpu
from jax.experimental.pallas import tpu_sc as plsc

NUM_FIELDS = 26
CARD = 100000
ROWS_PER_TABLE = CARD + 1
DIM = 50
BATCH = 16384

N = BATCH * NUM_FIELDS
NW = 32
NPW = N // NW
RPG = 128
G = NPW // RPG
LANES = 16


def _gather_kernel(table_hbm, x_hbm, out_hbm, idx_v, rows_v, sem):
    core = lax.axis_index("c")
    sub = lax.axis_index("s")
    wid = sub * 2 + core
    base = wid * NPW

    pltpu.sync_copy(x_hbm.at[pl.ds(base, NPW)], idx_v)

    def body(g, carry):
        off = pl.multiple_of(g * RPG, RPG)
        pltpu.async_copy(table_hbm.at[idx_v.at[pl.ds(off, RPG)]], rows_v,
                         sem).wait()
        pltpu.sync_copy(rows_v, out_hbm.at[pl.ds(base + off, RPG)])
        return carry

    lax.fori_loop(0, G, body, None)


@jax.jit
def _run(big_table, x_flat):
    mesh = plsc.VectorSubcoreMesh(core_axis_name="c", subcore_axis_name="s")
    f = functools.partial(
        pl.kernel,
        mesh=mesh,
        compiler_params=pltpu.CompilerParams(use_tc_tiling_on_sc=False),
        out_type=jax.ShapeDtypeStruct((N, DIM), jnp.float32),
        scratch_types=[
            pltpu.VMEM((NPW,), jnp.int32),
            pltpu.VMEM((RPG, DIM), jnp.float32),
            pltpu.SemaphoreType.DMA,
        ],
    )(_gather_kernel)
    return f(big_table, x_flat)


def kernel(x, tables):
    big_table = tables.reshape(NUM_FIELDS * ROWS_PER_TABLE, DIM)
    offs = jnp.arange(NUM_FIELDS, dtype=jnp.int32) * ROWS_PER_TABLE
    out = _run(big_table, (x + offs[None, :]).reshape(-1))
    return out.reshape(BATCH, NUM_FIELDS * DIM)
